# Initial kernel scaffold; baseline (speedup 1.0000x reference)
import jax
import jax.numpy as jnp
from jax.experimental import pallas as pl


def kernel(df, fc, gt):
    raise NotImplementedError("write your pallas kernel here")


# speedup vs baseline: 593.6087x; 593.6087x over previous
import jax
import jax.numpy as jnp
from jax.experimental import pallas as pl

_NCLS = 100
_ALP = 0.1


def _body(df_ref, fcp_ref, gtc_ref, gtr_ref, z_ref):
    df = df_ref[...]
    fcp = fcp_ref[...]
    gt_col = gtc_ref[...]
    gt_row = gtr_ref[...]
    bsz = df.shape[0]

    row_i = jax.lax.broadcasted_iota(jnp.int32, (bsz, bsz), 0)
    col_j = jax.lax.broadcasted_iota(jnp.int32, (bsz, bsz), 1)
    same = (gt_col == gt_row) & (col_j <= row_i)
    m = same.astype(jnp.float32)

    dot = lambda x, y: jax.lax.dot_general(
        x, y, (((1,), (0,)), ((), ())),
        precision=jax.lax.Precision.HIGHEST,
        preferred_element_type=jnp.float32)

    rank = jnp.sum(m, axis=1, keepdims=True)
    csum = dot(m, df)
    a = df - csum / rank
    b = dot(a, fcp.T)
    g = dot(df, fcp.T)

    cls = jax.lax.broadcasted_iota(jnp.int32, (bsz, 128), 1)
    onehot = (gt_col == cls).astype(jnp.float32)
    b_t = jnp.sum(b * onehot, axis=1, keepdims=True)
    g_t = jnp.sum(g * onehot, axis=1, keepdims=True)

    w = (rank - 1.0) / rank
    h = w * (b - b_t) ** 2
    s = dot(m, h)
    z_ref[...] = g - g_t + (0.5 * _ALP) * s / rank


def kernel(df, fc, gt):
    bsz, ndf = df.shape
    ncls = fc.shape[0]
    fcp = jnp.zeros((ndf, ndf), jnp.float32).at[:ncls].set(fc)
    gt_col = gt.reshape(bsz, 1)
    gt_row = gt.reshape(1, bsz)
    z = pl.pallas_call(
        _body,
        out_shape=jax.ShapeDtypeStruct((bsz, ndf), jnp.float32),
    )(df, fcp, gt_col, gt_row)
    return z[:, :ncls, None]
